# baseline (speedup 1.0000x reference)
import functools

import jax
import jax.numpy as jnp
from jax import lax
from jax.experimental import pallas as pl
from jax.experimental.pallas import tpu as pltpu
from jax.experimental.pallas import tpu_sc as plsc

N = 32768
E = 524288
B = 16
V = 128
H = 32
HM = 16

NUM_TILES = 16
EDGES_PER_TILE = E // NUM_TILES
NODES_PER_TILE = N // NUM_TILES
QROWS = N // 4
QELEMS = QROWS * V
SINK = QELEMS
GROUP = 1024
RING = 4
CHUNK = RING * GROUP
ZCHUNK = 8192


def _sc_count_body(labels_hbm, src_hbm, dst_hbm, cnt_hbm,
                   labels_v, src_c, dst_c, ring0, ring1, ring2, ring3,
                   ones_v, zbuf, cnt_sh, sem):
    ring = [ring0, ring1, ring2, ring3]
    c = lax.axis_index("c")
    s = lax.axis_index("s")

    pltpu.sync_copy(labels_hbm, labels_v)

    def _fill(i, _):
        ones_v[pl.ds(i * 16, 16)] = jnp.full((16,), 1.0, jnp.float32)
        return 0
    lax.fori_loop(0, GROUP // 16, _fill, 0)

    def _zfill(i, _):
        zbuf[pl.ds(i * 16, 16)] = jnp.zeros((16,), jnp.float32)
        return 0
    lax.fori_loop(0, ZCHUNK // 16, _zfill, 0)

    ebase = s * EDGES_PER_TILE
    nbase = s * NODES_PER_TILE
    lane = lax.iota(jnp.int32, 16)
    slice_elems = QELEMS // NUM_TILES

    for p in range(2):
        quarter = c * 2 + p
        qbase = quarter * QELEMS

        for z in range(slice_elems // ZCHUNK):
            pltpu.sync_copy(
                zbuf, cnt_sh.at[pl.ds(s * slice_elems + z * ZCHUNK, ZCHUNK)])
        @pl.when(s == 0)
        def _():
            pltpu.sync_copy(zbuf.at[pl.ds(0, 16)],
                            cnt_sh.at[pl.ds(QELEMS, 16)])
        plsc.subcore_barrier()

        def _edge_chunk(ch, _):
            pltpu.sync_copy(src_hbm.at[pl.ds(ebase + ch * CHUNK, CHUNK)],
                            src_c)
            pltpu.sync_copy(dst_hbm.at[pl.ds(ebase + ch * CHUNK, CHUNK)],
                            dst_c)
            copies = []
            for b in range(RING):
                for j in range(GROUP // 16):
                    base = b * GROUP + j * 16
                    sv = src_c[pl.ds(base, 16)]
                    dv = dst_c[pl.ds(base, 16)]
                    lab = plsc.load_gather(labels_v, [sv])
                    loc = dv * V + lab - qbase
                    ok = (loc >= 0) & (loc < QELEMS)
                    ring[b][pl.ds(j * 16, 16)] = jnp.where(ok, loc, SINK)
                copies.append(
                    pltpu.async_copy(ones_v, cnt_sh.at[ring[b]], sem,
                                     add=True))
            for cp in copies:
                cp.wait()
            return 0
        lax.fori_loop(0, EDGES_PER_TILE // CHUNK, _edge_chunk, 0)

        def _self_chunk(t, _):
            for j in range(GROUP // 16):
                off = t * GROUP + j * 16
                lab = labels_v[pl.ds(nbase + off, 16)]
                n = nbase + off + lane
                loc = n * V + lab - qbase
                ok = (loc >= 0) & (loc < QELEMS)
                ring[0][pl.ds(j * 16, 16)] = jnp.where(ok, loc, SINK)
            pltpu.async_copy(ones_v, cnt_sh.at[ring[0]], sem,
                             add=True).wait()
            return 0
        lax.fori_loop(0, NODES_PER_TILE // GROUP, _self_chunk, 0)
        plsc.subcore_barrier()

        pltpu.sync_copy(
            cnt_sh.at[pl.ds(s * slice_elems, slice_elems)],
            cnt_hbm.at[pl.ds(qbase + s * slice_elems, slice_elems)])
        plsc.subcore_barrier()


def _sc_count(labels, src, dst):
    mesh = plsc.VectorSubcoreMesh(core_axis_name="c", subcore_axis_name="s")
    return pl.kernel(
        _sc_count_body,
        out_type=jax.ShapeDtypeStruct((N * V,), jnp.float32),
        mesh=mesh,
        compiler_params=pltpu.CompilerParams(needs_layout_passes=False),
        scratch_types=[
            pltpu.VMEM((N,), jnp.int32),
            pltpu.VMEM((CHUNK,), jnp.int32),
            pltpu.VMEM((CHUNK,), jnp.int32),
            pltpu.VMEM((GROUP,), jnp.int32),
            pltpu.VMEM((GROUP,), jnp.int32),
            pltpu.VMEM((GROUP,), jnp.int32),
            pltpu.VMEM((GROUP,), jnp.int32),
            pltpu.VMEM((GROUP,), jnp.float32),
            pltpu.VMEM((ZCHUNK,), jnp.float32),
            pltpu.VMEM_SHARED((QELEMS + 16,), jnp.float32),
            pltpu.SemaphoreType.DMA,
        ],
    )(labels, src, dst)


def _tc_mlp_body(cnt_ref, emb_ref, wg_ref, w1_ref, b1_ref, w2_ref, b2_ref,
                 out_ref):
    m = jnp.dot(emb_ref[...], wg_ref[...],
                preferred_element_type=jnp.float32,
                precision=lax.Precision.HIGHEST)
    h = jnp.maximum(
        jnp.dot(cnt_ref[...], m, preferred_element_type=jnp.float32,
                precision=lax.Precision.HIGHEST), 0.0)
    t = jnp.maximum(
        jnp.dot(h, w1_ref[...], preferred_element_type=jnp.float32,
                precision=lax.Precision.HIGHEST) + b1_ref[...], 0.0)
    sval = jnp.sum(t * w2_ref[...], axis=1) + b2_ref[0, 0]
    out_ref[...] = sval.reshape(out_ref.shape)


def _tc_mlp(cnt, emb, wg, w1, b1r, w2r, b2r):
    blk = 2048
    grid = N // blk
    return pl.pallas_call(
        _tc_mlp_body,
        grid=(grid,),
        in_specs=[
            pl.BlockSpec((blk, V), lambda i: (i, 0)),
            pl.BlockSpec((V, H), lambda i: (0, 0)),
            pl.BlockSpec((H, H), lambda i: (0, 0)),
            pl.BlockSpec((H, HM), lambda i: (0, 0)),
            pl.BlockSpec((1, HM), lambda i: (0, 0)),
            pl.BlockSpec((1, HM), lambda i: (0, 0)),
            pl.BlockSpec((1, 1), lambda i: (0, 0)),
        ],
        out_specs=pl.BlockSpec((blk // 128, 128), lambda i: (i, 0)),
        out_shape=jax.ShapeDtypeStruct((N // 128, 128), jnp.float32),
    )(cnt, emb, wg, w1, b1r, w2r, b2r)


def _tc_seg_body(s_ref, seg_ref, gum_ref, out_ref):
    sval = s_ref[...]
    seg = seg_ref[...]
    gum = gum_ref[...]
    neg = jnp.float32(-jnp.inf)

    smax_node = jnp.zeros_like(sval)
    for b in range(B):
        m = seg == b
        smax_node = jnp.where(m, jnp.max(jnp.where(m, sval, neg)), smax_node)
    sh = sval - smax_node
    ex = jnp.exp(sh)
    logsum_node = jnp.zeros_like(sval)
    for b in range(B):
        m = seg == b
        logsum_node = jnp.where(
            m, jnp.log(jnp.sum(jnp.where(m, ex, 0.0))), logsum_node)
    logp = sh - logsum_node
    pert = logp + gum
    win_node = jnp.zeros_like(sval)
    for b in range(B):
        m = seg == b
        win_node = jnp.where(m, jnp.max(jnp.where(m, pert, neg)), win_node)
    wmask = pert == win_node
    iota16 = lax.broadcasted_iota(jnp.int32, (1, B), 1)
    res = jnp.zeros((1, B), jnp.float32)
    for b in range(B):
        m = seg == b
        val = jnp.sum(jnp.where(m & wmask, logp, 0.0))
        res = res + jnp.where(iota16 == b, val, 0.0)
    out_ref[...] = res


def _tc_seg(s2d, seg2d, gum2d):
    return pl.pallas_call(
        _tc_seg_body,
        out_shape=jax.ShapeDtypeStruct((1, B), jnp.float32),
    )(s2d, seg2d, gum2d)


def kernel(node_labels, edge_index, segment_ids, gumbel, emb, W_gcn, W1, b1,
           W2, b2):
    cnt = _sc_count(node_labels, edge_index[0], edge_index[1]).reshape(N, V)
    s2d = _tc_mlp(cnt, emb, W_gcn, W1, b1.reshape(1, HM), W2.reshape(1, HM),
                  b2.reshape(1, 1))
    out = _tc_seg(s2d, segment_ids.reshape(N // 128, 128),
                  gumbel.reshape(N // 128, 128))
    return out.reshape(B)


# speedup vs baseline: 4.4337x; 4.4337x over previous
import functools

import jax
import jax.numpy as jnp
from jax import lax
from jax.experimental import pallas as pl
from jax.experimental.pallas import tpu as pltpu
from jax.experimental.pallas import tpu_sc as plsc

N = 32768
E = 524288
B = 16
V = 128
H = 32
HM = 16

NUM_TILES = 16
EDGES_PER_TILE = E // NUM_TILES
NODES_PER_TILE = N // NUM_TILES
QROWS = N // 4
QELEMS = QROWS * V
SINK = QELEMS
GROUP = 512
CHUNK = 4096
SBUF = CHUNK + 16
ZCHUNK = 8192


def _sc_count_body(labels_hbm, src_hbm, dst_hbm, cnt_hbm,
                   labels_v, src_c, dst_c, sbuf, ones_v, zbuf, cnt_sh, sem):
    c = lax.axis_index("c")
    s = lax.axis_index("s")

    pltpu.sync_copy(labels_hbm, labels_v)

    def _fill(i, _):
        ones_v[pl.ds(i * 16, 16)] = jnp.full((16,), 1.0, jnp.float32)
        return 0
    lax.fori_loop(0, GROUP // 16, _fill, 0)

    def _zfill(i, _):
        zbuf[pl.ds(i * 16, 16)] = jnp.zeros((16,), jnp.float32)
        return 0
    lax.fori_loop(0, ZCHUNK // 16, _zfill, 0)

    ebase = s * EDGES_PER_TILE
    nbase = s * NODES_PER_TILE
    lane = lax.iota(jnp.int32, 16)
    slice_elems = QELEMS // NUM_TILES

    for p in range(2):
        quarter = c * 2 + p
        qbase = quarter * QELEMS

        for z in range(slice_elems // ZCHUNK):
            pltpu.sync_copy(
                zbuf, cnt_sh.at[pl.ds(s * slice_elems + z * ZCHUNK, ZCHUNK)])
        @pl.when(s == 0)
        def _():
            pltpu.sync_copy(zbuf.at[pl.ds(0, 16)],
                            cnt_sh.at[pl.ds(QELEMS, 16)])
        plsc.subcore_barrier()

        sink_vec = jnp.full((16,), SINK, jnp.int32)

        def _compress_fire(n_items, gen):
            def _pre(i, _):
                sbuf[pl.ds(i * 16, 16)] = sink_vec
                return 0
            lax.fori_loop(0, (n_items + 16) // 16, _pre, 0)

            def _step(j, off):
                loc, ok = gen(j)
                plsc.store_compressed(sbuf.at[pl.ds(off, 16)], loc, mask=ok)
                return off + plsc.all_reduce_population_count(ok)[0]
            t = lax.fori_loop(0, n_items // 16, _step, jnp.int32(0))
            for b in range(n_items // GROUP):
                @pl.when(t > b * GROUP)
                def _():
                    pltpu.async_copy(
                        ones_v, cnt_sh.at[sbuf.at[pl.ds(b * GROUP, GROUP)]],
                        sem, add=True)
            for b in range(n_items // GROUP):
                @pl.when(t > b * GROUP)
                def _():
                    pltpu.make_async_copy(
                        ones_v, cnt_sh.at[sbuf.at[pl.ds(b * GROUP, GROUP)]],
                        sem).wait()

        def _edge_chunk(ch, _):
            pltpu.sync_copy(src_hbm.at[pl.ds(ebase + ch * CHUNK, CHUNK)],
                            src_c)
            pltpu.sync_copy(dst_hbm.at[pl.ds(ebase + ch * CHUNK, CHUNK)],
                            dst_c)

            def _gen(j):
                sv = src_c[pl.ds(j * 16, 16)]
                dv = dst_c[pl.ds(j * 16, 16)]
                lab = plsc.load_gather(labels_v, [sv])
                loc = dv * V + lab - qbase
                ok = (loc >= 0) & (loc < QELEMS)
                return loc, ok
            _compress_fire(CHUNK, _gen)
            return 0
        lax.fori_loop(0, EDGES_PER_TILE // CHUNK, _edge_chunk, 0)

        def _gen_self(j):
            lab = labels_v[pl.ds(nbase + j * 16, 16)]
            n = nbase + j * 16 + lane
            loc = n * V + lab - qbase
            ok = (loc >= 0) & (loc < QELEMS)
            return loc, ok
        _compress_fire(NODES_PER_TILE, _gen_self)
        plsc.subcore_barrier()

        pltpu.sync_copy(
            cnt_sh.at[pl.ds(s * slice_elems, slice_elems)],
            cnt_hbm.at[pl.ds(qbase + s * slice_elems, slice_elems)])
        plsc.subcore_barrier()


def _sc_count(labels, src, dst):
    mesh = plsc.VectorSubcoreMesh(core_axis_name="c", subcore_axis_name="s")
    return pl.kernel(
        _sc_count_body,
        out_type=jax.ShapeDtypeStruct((N * V,), jnp.float32),
        mesh=mesh,
        compiler_params=pltpu.CompilerParams(needs_layout_passes=False),
        scratch_types=[
            pltpu.VMEM((N,), jnp.int32),
            pltpu.VMEM((CHUNK,), jnp.int32),
            pltpu.VMEM((CHUNK,), jnp.int32),
            pltpu.VMEM((SBUF,), jnp.int32),
            pltpu.VMEM((GROUP,), jnp.float32),
            pltpu.VMEM((ZCHUNK,), jnp.float32),
            pltpu.VMEM_SHARED((QELEMS + 16,), jnp.float32),
            pltpu.SemaphoreType.DMA,
        ],
    )(labels, src, dst)


def _tc_mlp_body(cnt_ref, emb_ref, wg_ref, w1_ref, b1_ref, w2_ref, b2_ref,
                 out_ref):
    m = jnp.dot(emb_ref[...], wg_ref[...],
                preferred_element_type=jnp.float32,
                precision=lax.Precision.HIGHEST)
    h = jnp.maximum(
        jnp.dot(cnt_ref[...], m, preferred_element_type=jnp.float32,
                precision=lax.Precision.HIGHEST), 0.0)
    t = jnp.maximum(
        jnp.dot(h, w1_ref[...], preferred_element_type=jnp.float32,
                precision=lax.Precision.HIGHEST) + b1_ref[...], 0.0)
    sval = jnp.sum(t * w2_ref[...], axis=1) + b2_ref[0, 0]
    out_ref[...] = sval.reshape(out_ref.shape)


def _tc_mlp(cnt, emb, wg, w1, b1r, w2r, b2r):
    blk = 2048
    grid = N // blk
    return pl.pallas_call(
        _tc_mlp_body,
        grid=(grid,),
        in_specs=[
            pl.BlockSpec((blk, V), lambda i: (i, 0)),
            pl.BlockSpec((V, H), lambda i: (0, 0)),
            pl.BlockSpec((H, H), lambda i: (0, 0)),
            pl.BlockSpec((H, HM), lambda i: (0, 0)),
            pl.BlockSpec((1, HM), lambda i: (0, 0)),
            pl.BlockSpec((1, HM), lambda i: (0, 0)),
            pl.BlockSpec((1, 1), lambda i: (0, 0)),
        ],
        out_specs=pl.BlockSpec((blk // 128, 128), lambda i: (i, 0)),
        out_shape=jax.ShapeDtypeStruct((N // 128, 128), jnp.float32),
    )(cnt, emb, wg, w1, b1r, w2r, b2r)


def _tc_seg_body(s_ref, seg_ref, gum_ref, out_ref):
    sval = s_ref[...]
    seg = seg_ref[...]
    gum = gum_ref[...]
    neg = jnp.float32(-jnp.inf)

    smax_node = jnp.zeros_like(sval)
    for b in range(B):
        m = seg == b
        smax_node = jnp.where(m, jnp.max(jnp.where(m, sval, neg)), smax_node)
    sh = sval - smax_node
    ex = jnp.exp(sh)
    logsum_node = jnp.zeros_like(sval)
    for b in range(B):
        m = seg == b
        logsum_node = jnp.where(
            m, jnp.log(jnp.sum(jnp.where(m, ex, 0.0))), logsum_node)
    logp = sh - logsum_node
    pert = logp + gum
    win_node = jnp.zeros_like(sval)
    for b in range(B):
        m = seg == b
        win_node = jnp.where(m, jnp.max(jnp.where(m, pert, neg)), win_node)
    wmask = pert == win_node
    iota16 = lax.broadcasted_iota(jnp.int32, (1, B), 1)
    res = jnp.zeros((1, B), jnp.float32)
    for b in range(B):
        m = seg == b
        val = jnp.sum(jnp.where(m & wmask, logp, 0.0))
        res = res + jnp.where(iota16 == b, val, 0.0)
    out_ref[...] = res


def _tc_seg(s2d, seg2d, gum2d):
    return pl.pallas_call(
        _tc_seg_body,
        out_shape=jax.ShapeDtypeStruct((1, B), jnp.float32),
    )(s2d, seg2d, gum2d)


def kernel(node_labels, edge_index, segment_ids, gumbel, emb, W_gcn, W1, b1,
           W2, b2):
    cnt = _sc_count(node_labels, edge_index[0], edge_index[1]).reshape(N, V)
    s2d = _tc_mlp(cnt, emb, W_gcn, W1, b1.reshape(1, HM), W2.reshape(1, HM),
                  b2.reshape(1, 1))
    out = _tc_seg(s2d, segment_ids.reshape(N // 128, 128),
                  gumbel.reshape(N // 128, 128))
    return out.reshape(B)


# speedup vs baseline: 5.4495x; 1.2291x over previous
import functools

import jax
import jax.numpy as jnp
from jax import lax
from jax.experimental import pallas as pl
from jax.experimental.pallas import tpu as pltpu
from jax.experimental.pallas import tpu_sc as plsc

N = 32768
E = 524288
B = 16
V = 128
H = 32
HM = 16

NUM_TILES = 16
EDGES_PER_TILE = E // NUM_TILES
NODES_PER_TILE = N // NUM_TILES
QROWS = N // 4
QELEMS = QROWS * V
SINK = QELEMS
GROUP = 512
CHUNK = 2048
SBUF = CHUNK + 16
SPILL = 12288
ZCHUNK = 2048


def _sc_count_body(labels_hbm, edge_hbm, cnt_hbm,
                   labels_v, eca, ecb, sbuf, spill, ones_v, zbuf,
                   cnt_sh, sem, sem_pf):
    c = lax.axis_index("c")
    s = lax.axis_index("s")

    pltpu.sync_copy(labels_hbm, labels_v)

    def _fill(i, _):
        ones_v[pl.ds(i * 16, 16)] = jnp.full((16,), 1.0, jnp.float32)
        return 0
    lax.fori_loop(0, GROUP // 16, _fill, 0)

    def _zfill(i, _):
        zbuf[pl.ds(i * 16, 16)] = jnp.zeros((16,), jnp.float32)
        return 0
    lax.fori_loop(0, ZCHUNK // 16, _zfill, 0)

    sink_vec = jnp.full((16,), SINK, jnp.int32)

    def _sinkfill(ref, n):
        def _body(i, _):
            ref[pl.ds(i * 16, 16)] = sink_vec
            return 0
        lax.fori_loop(0, n // 16, _body, 0)

    _sinkfill(spill, SPILL)

    ebase = s * EDGES_PER_TILE
    nbase = s * NODES_PER_TILE
    lane = lax.iota(jnp.int32, 16)
    slice_elems = QELEMS // NUM_TILES
    qbase_a = (c * 2) * QELEMS
    qbase_b = (c * 2 + 1) * QELEMS

    def _zero_slice():
        copies = [
            pltpu.async_copy(
                zbuf,
                cnt_sh.at[pl.ds(s * slice_elems + z * ZCHUNK, ZCHUNK)],
                sem)
            for z in range(slice_elems // ZCHUNK)
        ]
        for cp in copies:
            cp.wait()
        @pl.when(s == 0)
        def _():
            pltpu.sync_copy(zbuf.at[pl.ds(0, 16)],
                            cnt_sh.at[pl.ds(QELEMS, 16)])

    def _fire_groups(buf, nmax, t):
        for b in range(nmax // GROUP):
            @pl.when(t > b * GROUP)
            def _():
                pltpu.async_copy(
                    ones_v, cnt_sh.at[buf.at[pl.ds(b * GROUP, GROUP)]],
                    sem, add=True)
        for b in range(nmax // GROUP):
            @pl.when(t > b * GROUP)
            def _():
                pltpu.make_async_copy(
                    ones_v, cnt_sh.at[buf.at[pl.ds(b * GROUP, GROUP)]],
                    sem).wait()

    _zero_slice()
    plsc.subcore_barrier()

    def _ec_slice(ch):
        return edge_hbm.at[:, pl.ds(ebase + ch * CHUNK, CHUNK)]

    pltpu.async_copy(_ec_slice(0), eca, sem_pf)
    nchunks = EDGES_PER_TILE // CHUNK

    def _sweep(gen, n_items, off_b0):
        _sinkfill(sbuf, SBUF)

        def _step(j, carry):
            off_a, off_b = carry
            loc, ok_a, ok_b = gen(j)
            plsc.store_compressed(sbuf.at[pl.ds(off_a, 16)], loc, mask=ok_a)
            plsc.store_compressed(spill.at[pl.ds(off_b, 16)], loc - QELEMS,
                                  mask=ok_b)
            return (off_a + plsc.all_reduce_population_count(ok_a)[0],
                    off_b + plsc.all_reduce_population_count(ok_b)[0])
        t_a, off_b = lax.fori_loop(0, n_items // 16, _step,
                                   (jnp.int32(0), off_b0))
        _fire_groups(sbuf, n_items, t_a)
        return off_b

    def _edge_pair(g, off_b):
        for half, (cur, other) in enumerate(((eca, ecb), (ecb, eca))):
            ch = g * 2 + half
            pltpu.make_async_copy(_ec_slice(ch), cur, sem_pf).wait()
            @pl.when(ch + 1 < nchunks)
            def _():
                pltpu.async_copy(_ec_slice(ch + 1), other, sem_pf)

            def _gen(j):
                sv = cur[0, pl.ds(j * 16, 16)]
                dv = cur[1, pl.ds(j * 16, 16)]
                lab = plsc.load_gather(labels_v, [sv])
                loc = dv * V + lab - qbase_a
                ok_a = (loc >= 0) & (loc < QELEMS)
                ok_b = (loc >= QELEMS) & (loc < 2 * QELEMS)
                return loc, ok_a, ok_b
            off_b = _sweep(_gen, CHUNK, off_b)
        return off_b

    off_b = lax.fori_loop(0, nchunks // 2, _edge_pair, jnp.int32(0))

    def _gen_self(j):
        lab = labels_v[pl.ds(nbase + j * 16, 16)]
        n = nbase + j * 16 + lane
        loc = n * V + lab - qbase_a
        ok_a = (loc >= 0) & (loc < QELEMS)
        ok_b = (loc >= QELEMS) & (loc < 2 * QELEMS)
        return loc, ok_a, ok_b
    off_b = _sweep(_gen_self, NODES_PER_TILE, off_b)
    plsc.subcore_barrier()

    pltpu.sync_copy(
        cnt_sh.at[pl.ds(s * slice_elems, slice_elems)],
        cnt_hbm.at[pl.ds(qbase_a + s * slice_elems, slice_elems)])

    _zero_slice()
    plsc.subcore_barrier()
    _fire_groups(spill, SPILL, off_b)
    plsc.subcore_barrier()
    pltpu.sync_copy(
        cnt_sh.at[pl.ds(s * slice_elems, slice_elems)],
        cnt_hbm.at[pl.ds(qbase_b + s * slice_elems, slice_elems)])


def _sc_count(labels, edge_index):
    mesh = plsc.VectorSubcoreMesh(core_axis_name="c", subcore_axis_name="s")
    return pl.kernel(
        _sc_count_body,
        out_type=jax.ShapeDtypeStruct((N * V,), jnp.float32),
        mesh=mesh,
        compiler_params=pltpu.CompilerParams(needs_layout_passes=False),
        scratch_types=[
            pltpu.VMEM((N,), jnp.int32),
            pltpu.VMEM((2, CHUNK), jnp.int32),
            pltpu.VMEM((2, CHUNK), jnp.int32),
            pltpu.VMEM((SBUF,), jnp.int32),
            pltpu.VMEM((SPILL,), jnp.int32),
            pltpu.VMEM((GROUP,), jnp.float32),
            pltpu.VMEM((ZCHUNK,), jnp.float32),
            pltpu.VMEM_SHARED((QELEMS + 16,), jnp.float32),
            pltpu.SemaphoreType.DMA,
            pltpu.SemaphoreType.DMA,
        ],
    )(labels, edge_index)


def _tc_mlp_body(cnt_ref, emb_ref, wg_ref, w1_ref, b1_ref, w2_ref, b2_ref,
                 out_ref):
    m = jnp.dot(emb_ref[...], wg_ref[...],
                preferred_element_type=jnp.float32,
                precision=lax.Precision.HIGHEST)
    h = jnp.maximum(
        jnp.dot(cnt_ref[...], m,
                preferred_element_type=jnp.float32,
                precision=lax.Precision.HIGHEST), 0.0)
    t = jnp.maximum(
        jnp.dot(h, w1_ref[...], preferred_element_type=jnp.float32,
                precision=lax.Precision.HIGHEST) + b1_ref[...], 0.0)
    sval = jnp.sum(t * w2_ref[...], axis=1) + b2_ref[0, 0]
    out_ref[...] = sval.reshape(out_ref.shape)


def _tc_mlp(cnt, emb, wg, w1, b1r, w2r, b2r):
    blk = 2048
    grid = N // blk
    return pl.pallas_call(
        _tc_mlp_body,
        grid=(grid,),
        in_specs=[
            pl.BlockSpec((blk, V), lambda i: (i, 0)),
            pl.BlockSpec((V, H), lambda i: (0, 0)),
            pl.BlockSpec((H, H), lambda i: (0, 0)),
            pl.BlockSpec((H, HM), lambda i: (0, 0)),
            pl.BlockSpec((1, HM), lambda i: (0, 0)),
            pl.BlockSpec((1, HM), lambda i: (0, 0)),
            pl.BlockSpec((1, 1), lambda i: (0, 0)),
        ],
        out_specs=pl.BlockSpec((blk // 128, 128), lambda i: (i, 0)),
        out_shape=jax.ShapeDtypeStruct((N // 128, 128), jnp.float32),
    )(cnt, emb, wg, w1, b1r, w2r, b2r)


def _tc_seg_body(s_ref, seg_ref, gum_ref, out_ref):
    sval = s_ref[...]
    seg = seg_ref[...]
    gum = gum_ref[...]
    neg = jnp.float32(-jnp.inf)

    smax_node = jnp.zeros_like(sval)
    for b in range(B):
        m = seg == b
        smax_node = jnp.where(m, jnp.max(jnp.where(m, sval, neg)), smax_node)
    sh = sval - smax_node
    ex = jnp.exp(sh)
    logsum_node = jnp.zeros_like(sval)
    for b in range(B):
        m = seg == b
        logsum_node = jnp.where(
            m, jnp.log(jnp.sum(jnp.where(m, ex, 0.0))), logsum_node)
    logp = sh - logsum_node
    pert = logp + gum
    win_node = jnp.zeros_like(sval)
    for b in range(B):
        m = seg == b
        win_node = jnp.where(m, jnp.max(jnp.where(m, pert, neg)), win_node)
    wmask = pert == win_node
    iota16 = lax.broadcasted_iota(jnp.int32, (1, B), 1)
    res = jnp.zeros((1, B), jnp.float32)
    for b in range(B):
        m = seg == b
        val = jnp.sum(jnp.where(m & wmask, logp, 0.0))
        res = res + jnp.where(iota16 == b, val, 0.0)
    out_ref[...] = res


def _tc_seg(s2d, seg2d, gum2d):
    return pl.pallas_call(
        _tc_seg_body,
        out_shape=jax.ShapeDtypeStruct((1, B), jnp.float32),
    )(s2d, seg2d, gum2d)


def kernel(node_labels, edge_index, segment_ids, gumbel, emb, W_gcn, W1, b1,
           W2, b2):
    cnt = _sc_count(node_labels, edge_index).reshape(N, V)
    s2d = _tc_mlp(cnt, emb, W_gcn, W1, b1.reshape(1, HM), W2.reshape(1, HM),
                  b2.reshape(1, 1))
    out = _tc_seg(s2d, segment_ids.reshape(N // 128, 128),
                  gumbel.reshape(N // 128, 128))
    return out.reshape(B)
